# baseline (speedup 1.0000x reference)
import jax
import jax.numpy as jnp
from jax import lax
from jax.experimental import pallas as pl
from jax.experimental.pallas import tpu as pltpu
from jax.experimental.pallas import tpu_sc as plsc

N = 32768
R = 64
S = 8
IN_CH = 256
MID_CH = 256
C = MID_CH // S
OUT_CH = 128
ROWS = N * S

NC, NS = 2, 16
NW = NC * NS
PW = N // NW
NPT = N // NS
BATCH = 256
NBATCH = (ROWS // NW) // BATCH
DMA_ROWS = 128
NDMA = BATCH // DMA_ROWS
NGRP = BATCH // S // 16
NBINS = R * R
BPT = NBINS // NS
PLEN = NBINS + 16


def _tc_body(x_ref, wm_ref, wo_ref, o_ref, wb_v):
    BM = x_ref.shape[0]

    @pl.when(pl.program_id(0) == 0)
    def _():
        wb_v[...] = jnp.zeros((MID_CH, S * OUT_CH), jnp.float32)
        for l in range(S):
            wb_v[l * C:(l + 1) * C, l * OUT_CH:(l + 1) * OUT_CH] = wo_ref[...]

    hb = jnp.dot(x_ref[...], wm_ref[...], preferred_element_type=jnp.float32)
    od = jnp.dot(hb, wb_v[...], preferred_element_type=jnp.float32)
    o_ref[...] = od.reshape(BM, S, OUT_CH)


def _tc_dense(x, wm, wo):
    BM = 1024
    return pl.pallas_call(
        _tc_body,
        grid=(N // BM,),
        in_specs=[
            pl.BlockSpec((BM, IN_CH), lambda m: (m, 0)),
            pl.BlockSpec((IN_CH, MID_CH), lambda m: (0, 0)),
            pl.BlockSpec((C, OUT_CH), lambda m: (0, 0)),
        ],
        out_specs=pl.BlockSpec((BM, S, OUT_CH), lambda m: (m, 0, 0)),
        out_shape=jax.ShapeDtypeStruct((N, S, OUT_CH), jnp.float32),
        scratch_shapes=[pltpu.VMEM((MID_CH, S * OUT_CH), jnp.float32)],
    )(x, wm, wo)


def _sc_body(ijk_ref, od_ref, out_ref,
             hist_v, ijk_v, comb_v, pfx_v, tot16_v, totb_v, pout_v, p_v,
             q_v, rows2_v, idx2_v,
             hists_sh, tot_sh, p_sh, g_sem, w_sem):
    c = lax.axis_index("c")
    s = lax.axis_index("s")
    w = s * NC + c
    lanes = lax.iota(jnp.int32, 16)
    zeros16 = jnp.zeros((16,), jnp.int32)
    ones16 = jnp.ones((16,), jnp.int32)

    def zero_body(t, carry):
        hist_v[pl.ds(t * 16, 16)] = zeros16
        return carry
    lax.fori_loop(0, NBINS // 16, zero_body, 0)

    pltpu.sync_copy(ijk_ref.at[pl.ds(s * (NPT * 3), NPT * 3)],
                    ijk_v.at[pl.ds(0, NPT * 3)])

    def hist_body(t, carry):
        base3 = (t * 16 + lanes) * 3
        iv = plsc.load_gather(ijk_v, [base3])
        jv = plsc.load_gather(ijk_v, [base3 + 1])
        plsc.addupdate_scatter(hist_v, [iv * R + jv], ones16)
        return carry
    lax.fori_loop(0, NPT // 16, hist_body, 0)

    pltpu.sync_copy(hist_v, hists_sh.at[s])
    plsc.subcore_barrier()

    for r in range(NS):
        pltpu.sync_copy(hists_sh.at[r, pl.ds(s * BPT, BPT)], comb_v.at[r])

    def pref_body(t, tot):
        v = comb_v[0, pl.ds(t * 16, 16)]
        for r in range(1, NS):
            v = v + comb_v[r, pl.ds(t * 16, 16)]
        incl = plsc.cumsum(v)
        pfx_v[pl.ds(t * 16, 16)] = incl - v + tot
        return tot + jnp.sum(v)
    tile_total = lax.fori_loop(0, BPT // 16, pref_body, 0)

    totb_v[...] = jnp.full((16,), tile_total, jnp.int32)
    pltpu.sync_copy(totb_v, tot_sh.at[s])
    plsc.subcore_barrier()

    pltpu.sync_copy(tot_sh, tot16_v)
    tvec = plsc.load_gather(tot16_v, [lanes, zeros16])
    offs = jnp.sum(jnp.where(lanes < s, tvec, 0))

    def poff_body(t, carry):
        pout_v[pl.ds(t * 16, 16)] = pfx_v[pl.ds(t * 16, 16)] + offs
        return carry
    lax.fori_loop(0, BPT // 16, poff_body, 0)
    pltpu.sync_copy(pout_v, p_sh.at[pl.ds(s * BPT, BPT)])

    @pl.when(s == NS - 1)
    def _():
        totb_v[...] = jnp.full((16,), N, jnp.int32)
        pltpu.sync_copy(totb_v, p_sh.at[pl.ds(NBINS, 16)])
    plsc.subcore_barrier()

    pltpu.sync_copy(p_sh, p_v)

    def q_body(t, carry):
        qi = jnp.minimum((t * 16 + lanes) * R, NBINS)
        q_v[pl.ds(t * 16, 16)] = plsc.load_gather(p_v, [qi]) * 8
        return carry
    lax.fori_loop(0, 5, q_body, 0)

    wbase = w * (NBATCH * BATCH)

    def idx_and_gather(b):
        mod2 = lax.rem(b, 2)
        rowbase = wbase + b * BATCH
        for g in range(BATCH // 16):
            r = rowbase + g * 16 + lanes
            pos = zeros16
            for bit in (32, 16, 8, 4, 2, 1):
                cand = pos + bit
                qv = plsc.load_gather(q_v, [cand])
                pos = jnp.where(qv <= r, cand, pos)
            qi = plsc.load_gather(q_v, [pos])
            qi1 = plsc.load_gather(q_v, [pos + 1])
            ai = qi >> 3
            ci4 = (qi1 - qi) >> 1
            r1 = r - qi
            di = jnp.where(r1 >= ci4, 1, 0)
            r2 = r1 - di * ci4
            pos2 = zeros16
            ibase = pos * R
            for bit in (32, 16, 8, 4, 2, 1):
                cand = pos2 + bit
                pv = plsc.load_gather(p_v, [ibase + cand])
                pos2 = jnp.where(4 * (pv - ai) <= r2, cand, pos2)
            key = ibase + pos2
            aij = plsc.load_gather(p_v, [key])
            bij = plsc.load_gather(p_v, [key + 1])
            r3 = r2 - 4 * (aij - ai)
            cij2 = 2 * (bij - aij)
            dj = jnp.where(r3 >= cij2, 1, 0)
            r4 = r3 - dj * cij2
            src_row = 8 * (aij + (r4 >> 1)) + 4 * di + 2 * dj + (r4 & 1)
            idx2_v[mod2, g // 8, pl.ds((g % 8) * 16, 16)] = src_row
        for j in range(NDMA):
            pltpu.async_copy(od_ref.at[idx2_v.at[mod2, j]],
                             rows2_v.at[mod2, pl.ds(j * DMA_ROWS, DMA_ROWS)],
                             g_sem)

    def drain_g():
        for j in range(NDMA):
            pltpu.make_async_copy(
                od_ref.at[pl.ds(0, DMA_ROWS)],
                rows2_v.at[0, pl.ds(0, DMA_ROWS)], g_sem).wait()

    def drain_w():
        pltpu.make_async_copy(od_ref.at[pl.ds(0, BATCH)],
                              rows2_v.at[0], w_sem).wait()

    def fire_w(b):
        mod2 = lax.rem(b, 2)
        rowbase = pl.multiple_of(wbase + b * BATCH, 8)
        pltpu.async_copy(rows2_v.at[mod2], out_ref.at[pl.ds(rowbase, BATCH)],
                         w_sem)

    idx_and_gather(0)

    def sbody(b, carry):
        @pl.when(b >= 1)
        def _():
            drain_w()
        @pl.when(b + 1 < NBATCH)
        def _():
            idx_and_gather(b + 1)
        drain_g()
        fire_w(b)
        return carry
    lax.fori_loop(0, NBATCH, sbody, 0)
    drain_w()


def _sc_scatter(ijk_flat, od):
    mesh = plsc.VectorSubcoreMesh(core_axis_name="c", subcore_axis_name="s",
                                  num_cores=NC, num_subcores=NS)
    f = pl.kernel(
        _sc_body,
        out_type=jax.ShapeDtypeStruct((ROWS, OUT_CH), jnp.float32),
        mesh=mesh,
        scratch_types=[
            pltpu.VMEM((NBINS,), jnp.int32),
            pltpu.VMEM((NPT * 3,), jnp.int32),
            pltpu.VMEM((NS, BPT), jnp.int32),
            pltpu.VMEM((BPT,), jnp.int32),
            pltpu.VMEM((NS, 16), jnp.int32),
            pltpu.VMEM((16,), jnp.int32),
            pltpu.VMEM((BPT,), jnp.int32),
            pltpu.VMEM((PLEN,), jnp.int32),
            pltpu.VMEM((80,), jnp.int32),
            pltpu.VMEM((2, BATCH, OUT_CH), jnp.float32),
            pltpu.VMEM((2, NDMA, DMA_ROWS), jnp.int32),
            pltpu.VMEM_SHARED((NS, NBINS), jnp.int32),
            pltpu.VMEM_SHARED((NS, 16), jnp.int32),
            pltpu.VMEM_SHARED((PLEN,), jnp.int32),
            pltpu.SemaphoreType.DMA,
            pltpu.SemaphoreType.DMA,
        ],
        compiler_params=pltpu.CompilerParams(needs_layout_passes=False),
    )
    return f(ijk_flat, od)


def kernel(x_data, ijk, W_mid, W_out):
    ijk_flat = ijk.reshape(-1).astype(jnp.int32)
    od3 = _tc_dense(x_data, W_mid, W_out)
    return _sc_scatter(ijk_flat, od3.reshape(ROWS, OUT_CH))


# speedup vs baseline: 1.1371x; 1.1371x over previous
import jax
import jax.numpy as jnp
from jax import lax
from jax.experimental import pallas as pl
from jax.experimental.pallas import tpu as pltpu
from jax.experimental.pallas import tpu_sc as plsc

N = 32768
R = 64
S = 8
IN_CH = 256
MID_CH = 256
C = MID_CH // S
OUT_CH = 128
ROWS = N * S

NC, NS = 2, 16
NW = NC * NS
PW = N // NW
NPT = N // NS
BATCH = 256
NBATCH = (ROWS // NW) // BATCH
DMA_ROWS = 128
NDMA = BATCH // DMA_ROWS
NGRP = BATCH // S // 16
NBINS = R * R
BPT = NBINS // NS
PLEN = NBINS + 16


def _tc_body(x_ref, wm_ref, wo_ref, o_ref, wb_v):
    BM = x_ref.shape[0]

    @pl.when(pl.program_id(0) == 0)
    def _():
        wb_v[...] = jnp.zeros((MID_CH, S * OUT_CH), jnp.float32)
        for l in range(S):
            wb_v[l * C:(l + 1) * C, l * OUT_CH:(l + 1) * OUT_CH] = wo_ref[...]

    hb = jnp.dot(x_ref[...], wm_ref[...], preferred_element_type=jnp.float32)
    od = jnp.dot(hb, wb_v[...], preferred_element_type=jnp.float32)
    o_ref[...] = od.reshape(BM, S, OUT_CH)


def _tc_dense(x, wm, wo):
    BM = 1024
    return pl.pallas_call(
        _tc_body,
        grid=(N // BM,),
        in_specs=[
            pl.BlockSpec((BM, IN_CH), lambda m: (m, 0)),
            pl.BlockSpec((IN_CH, MID_CH), lambda m: (0, 0)),
            pl.BlockSpec((C, OUT_CH), lambda m: (0, 0)),
        ],
        out_specs=pl.BlockSpec((BM, S, OUT_CH), lambda m: (m, 0, 0)),
        out_shape=jax.ShapeDtypeStruct((N, S, OUT_CH), jnp.float32),
        scratch_shapes=[pltpu.VMEM((MID_CH, S * OUT_CH), jnp.float32)],
    )(x, wm, wo)


def _sc_body(ii_ref, jj_ref, od_ref, out_ref,
             hist_v, ii_v, jj_v, comb_v, pfx_v, tot16_v, totb_v, pout_v, p_v,
             rows_a, rows_b, rows_c, idx_a, idx_b, idx_c,
             hists_sh, tot_sh, p_sh, in_sem, sc_sem):
    c = lax.axis_index("c")
    s = lax.axis_index("s")
    w = s * NC + c
    lanes = lax.iota(jnp.int32, 16)
    zeros16 = jnp.zeros((16,), jnp.int32)
    ones16 = jnp.ones((16,), jnp.int32)

    def zero_body(t, carry):
        hist_v[pl.ds(t * 16, 16)] = zeros16
        return carry
    lax.fori_loop(0, NBINS // 16, zero_body, 0)

    pltpu.sync_copy(ii_ref.at[pl.ds(s * NPT, NPT)], ii_v)
    pltpu.sync_copy(jj_ref.at[pl.ds(s * NPT, NPT)], jj_v)

    def hist_body(t, carry):
        iv = ii_v[pl.ds(t * 16, 16)]
        jv = jj_v[pl.ds(t * 16, 16)]
        plsc.addupdate_scatter(hist_v, [iv * R + jv], ones16)
        return carry
    lax.fori_loop(0, NPT // 16, hist_body, 0)

    pltpu.sync_copy(hist_v, hists_sh.at[s])
    plsc.subcore_barrier()

    for r in range(NS):
        pltpu.sync_copy(hists_sh.at[r, pl.ds(s * BPT, BPT)], comb_v.at[r])

    def pref_body(t, tot):
        v = comb_v[0, pl.ds(t * 16, 16)]
        for r in range(1, NS):
            v = v + comb_v[r, pl.ds(t * 16, 16)]
        incl = plsc.cumsum(v)
        pfx_v[pl.ds(t * 16, 16)] = incl - v + tot
        return tot + jnp.sum(v)
    tile_total = lax.fori_loop(0, BPT // 16, pref_body, 0)

    totb_v[...] = jnp.full((16,), tile_total, jnp.int32)
    pltpu.sync_copy(totb_v, tot_sh.at[s])
    plsc.subcore_barrier()

    pltpu.sync_copy(tot_sh, tot16_v)
    tvec = plsc.load_gather(tot16_v, [lanes, zeros16])
    offs = jnp.sum(jnp.where(lanes < s, tvec, 0))

    def poff_body(t, carry):
        pout_v[pl.ds(t * 16, 16)] = pfx_v[pl.ds(t * 16, 16)] + offs
        return carry
    lax.fori_loop(0, BPT // 16, poff_body, 0)
    pltpu.sync_copy(pout_v, p_sh.at[pl.ds(s * BPT, BPT)])

    @pl.when(s == NS - 1)
    def _():
        totb_v[...] = jnp.full((16,), N, jnp.int32)
        pltpu.sync_copy(totb_v, p_sh.at[pl.ds(NBINS, 16)])
    plsc.subcore_barrier()

    pltpu.sync_copy(p_sh, p_v)

    pltpu.sync_copy(ii_ref.at[pl.ds(w * PW, PW)], ii_v.at[pl.ds(0, PW)])
    pltpu.sync_copy(jj_ref.at[pl.ds(w * PW, PW)], jj_v.at[pl.ds(0, PW)])

    rows_bufs = (rows_a, rows_b, rows_c)
    idx_bufs = (idx_a, idx_b, idx_c)

    def start_in(b):
        rowbase = (w * NBATCH + b) * BATCH
        return pltpu.async_copy(od_ref.at[pl.ds(rowbase, BATCH)],
                                rows_bufs[b % 3], in_sem)

    in_descs = [None] * NBATCH
    sc_descs = [None] * NBATCH
    in_descs[0] = start_in(0)
    for b in range(NBATCH):
        in_descs[b].wait()
        if b >= 2:
            for d in sc_descs[b - 2]:
                d.wait()
        if b + 1 < NBATCH:
            in_descs[b + 1] = start_in(b + 1)
        idx_v = idx_bufs[b % 3]
        for g in range(NGRP):
            pgb = b * (BATCH // S) + g * 16
            pg = pgb + lanes
            iv = ii_v[pl.ds(pgb, 16)]
            jv = jj_v[pl.ds(pgb, 16)]
            key = iv * R + jv
            i64 = iv * R
            Ai = plsc.load_gather(p_v, [i64])
            Bi = plsc.load_gather(p_v, [i64 + R])
            Aij = plsc.load_gather(p_v, [key])
            Bij = plsc.load_gather(p_v, [key + 1])
            p = w * PW + pg
            base0 = 8 * Ai + 4 * (Aij - Ai) + 2 * (p - Aij)
            ci4 = 4 * (Bi - Ai)
            cij2 = 2 * (Bij - Aij)
            cols = lanes * 8
            for l in range(S):
                di, dj, dk = (l >> 2) & 1, (l >> 1) & 1, l & 1
                dst = base0 + di * ci4 + dj * cij2 + dk
                plsc.store_scatter(idx_v, [jnp.full((16,), g, jnp.int32),
                                           cols + l], dst)
        sc_descs[b] = [
            pltpu.async_copy(
                rows_bufs[b % 3].at[pl.ds(j * DMA_ROWS, DMA_ROWS)],
                out_ref.at[idx_v.at[j]],
                sc_sem)
            for j in range(NDMA)
        ]
    for d in sc_descs[NBATCH - 2] + sc_descs[NBATCH - 1]:
        d.wait()


def _sc_scatter(ii, jj, od):
    mesh = plsc.VectorSubcoreMesh(core_axis_name="c", subcore_axis_name="s",
                                  num_cores=NC, num_subcores=NS)
    f = pl.kernel(
        _sc_body,
        out_type=jax.ShapeDtypeStruct((ROWS, OUT_CH), jnp.float32),
        mesh=mesh,
        scratch_types=[
            pltpu.VMEM((NBINS,), jnp.int32),
            pltpu.VMEM((NPT,), jnp.int32),
            pltpu.VMEM((NPT,), jnp.int32),
            pltpu.VMEM((NS, BPT), jnp.int32),
            pltpu.VMEM((BPT,), jnp.int32),
            pltpu.VMEM((NS, 16), jnp.int32),
            pltpu.VMEM((16,), jnp.int32),
            pltpu.VMEM((BPT,), jnp.int32),
            pltpu.VMEM((PLEN,), jnp.int32),
            pltpu.VMEM((BATCH, OUT_CH), jnp.float32),
            pltpu.VMEM((BATCH, OUT_CH), jnp.float32),
            pltpu.VMEM((BATCH, OUT_CH), jnp.float32),
            pltpu.VMEM((NDMA, DMA_ROWS), jnp.int32),
            pltpu.VMEM((NDMA, DMA_ROWS), jnp.int32),
            pltpu.VMEM((NDMA, DMA_ROWS), jnp.int32),
            pltpu.VMEM_SHARED((NS, NBINS), jnp.int32),
            pltpu.VMEM_SHARED((NS, 16), jnp.int32),
            pltpu.VMEM_SHARED((PLEN,), jnp.int32),
            pltpu.SemaphoreType.DMA,
            pltpu.SemaphoreType.DMA,
        ],
        compiler_params=pltpu.CompilerParams(needs_layout_passes=False),
    )
    return f(ii, jj, od)


def kernel(x_data, ijk, W_mid, W_out):
    ii = ijk[:, 0].astype(jnp.int32)
    jj = ijk[:, 1].astype(jnp.int32)
    od3 = _tc_dense(x_data, W_mid, W_out)
    return _sc_scatter(ii, jj, od3.reshape(ROWS, OUT_CH))
